# Initial kernel scaffold; baseline (speedup 1.0000x reference)
import jax
import jax.numpy as jnp
from jax.experimental import pallas as pl


def kernel(x, edge_index, W1, b1, W2, b2, W3, b3, att, Wo, bo):
    raise NotImplementedError("write your pallas kernel here")


# speedup vs baseline: 11.4916x; 11.4916x over previous
import dataclasses
import functools
import math

import jax
import jax.numpy as jnp
from jax import lax
from jax.experimental import pallas as pl
from jax.experimental.pallas import tpu as pltpu
from jax.experimental.pallas import tpu_sc as plsc

NC, NS, LANES = 2, 16, 16
NW = NC * NS


def _sc_mesh():
    return plsc.VectorSubcoreMesh(
        core_axis_name="c", subcore_axis_name="s", num_cores=NC, num_subcores=NS
    )


def _sc_params():
    cp = pltpu.CompilerParams()
    if "needs_layout_passes" in pltpu.CompilerParams.__dataclass_fields__:
        cp = dataclasses.replace(cp, needs_layout_passes=False)
    return cp


def _deg_partials(dst, n):
    e = dst.shape[0]
    epw = e // NW
    steps = epw // LANES
    zsteps = n // LANES

    @functools.partial(
        pl.kernel,
        out_type=jax.ShapeDtypeStruct((NW * n,), jnp.float32),
        mesh=_sc_mesh(),
        compiler_params=_sc_params(),
        scratch_types=[
            pltpu.VMEM((epw,), jnp.int32),
            pltpu.VMEM((n,), jnp.float32),
        ],
    )
    def k(dst_hbm, out_hbm, didx, acc):
        c = lax.axis_index("c")
        s = lax.axis_index("s")
        wid = s * NC + c

        @pl.loop(0, zsteps)
        def _(j):
            acc[pl.ds(j * LANES, LANES)] = jnp.zeros((LANES,), jnp.float32)

        pltpu.sync_copy(dst_hbm.at[pl.ds(wid * epw, epw)], didx)
        ones = jnp.ones((LANES,), jnp.float32)

        @pl.loop(0, steps)
        def _(j):
            idx = didx[pl.ds(j * LANES, LANES)]
            plsc.addupdate_scatter(acc, [idx], ones)

        pltpu.sync_copy(acc, out_hbm.at[pl.ds(wid * n, n)])

    return k(dst).reshape(NW, n)


def _edge_scatter(g, src, dst):
    n, d = g.shape
    e = src.shape[0]
    epw = e // NW
    chunk = 80
    nch = epw // chunk
    rps = (n // NS) // 8 * 8
    rem = n - NS * rps

    @functools.partial(
        pl.kernel,
        out_type=jax.ShapeDtypeStruct((NC, n, d), jnp.float32),
        mesh=_sc_mesh(),
        compiler_params=_sc_params(),
        scratch_types=[
            pltpu.VMEM((chunk,), jnp.int32),
            pltpu.VMEM((chunk,), jnp.int32),
            pltpu.VMEM((chunk, d), jnp.float32),
            pltpu.VMEM_SHARED((n, d), jnp.float32),
            pltpu.SemaphoreType.DMA,
        ],
    )
    def k(g_hbm, src_hbm, dst_hbm, out_hbm, sidx, didx, rows, acc, sem):
        c = lax.axis_index("c")
        s = lax.axis_index("s")
        wid = s * NC + c

        pltpu.sync_copy(g_hbm.at[pl.ds(s * rps, rps)], acc.at[pl.ds(s * rps, rps)])
        if rem:
            @pl.when(s == 0)
            def _():
                pltpu.sync_copy(
                    g_hbm.at[pl.ds(NS * rps, rem)], acc.at[pl.ds(NS * rps, rem)]
                )
        plsc.subcore_barrier()

        @pl.loop(0, nch)
        def _(i):
            base = wid * epw + i * chunk
            pltpu.sync_copy(src_hbm.at[pl.ds(base, chunk)], sidx)
            pltpu.sync_copy(dst_hbm.at[pl.ds(base, chunk)], didx)
            pltpu.async_copy(g_hbm.at[sidx], rows, sem).wait()
            pltpu.sync_copy(rows, acc.at[didx], add=True)

        plsc.subcore_barrier()
        pltpu.sync_copy(
            acc.at[pl.ds(s * rps, rps)], out_hbm.at[c].at[pl.ds(s * rps, rps)]
        )
        if rem:
            @pl.when(s == 0)
            def _():
                pltpu.sync_copy(
                    acc.at[pl.ds(NS * rps, rem)],
                    out_hbm.at[c].at[pl.ds(NS * rps, rem)],
                )

    return k(g, src, dst)


def _tc_first(x, w, b2, dinv2):
    n = x.shape[0]
    dh = w.shape[1]

    def body(x_ref, w_ref, b_ref, dinv_ref, g_ref):
        hw = jnp.dot(x_ref[...], w_ref[...], preferred_element_type=jnp.float32)
        g_ref[...] = (hw + b_ref[...]) * dinv_ref[...]

    return pl.pallas_call(
        body, out_shape=jax.ShapeDtypeStruct((n, dh), jnp.float32)
    )(x, w, b2, dinv2)


def _tc_mid(s0, s1, g_prev, dinv2, w, b2):
    n, d = g_prev.shape
    dh = w.shape[1]

    def body(s0_ref, s1_ref, g_ref, dinv_ref, w_ref, b_ref, h_ref, gn_ref):
        pre = s0_ref[...] + s1_ref[...] - g_ref[...]
        h = jnp.maximum(pre * dinv_ref[...], 0.0)
        h_ref[...] = h
        hw = jnp.dot(h, w_ref[...], preferred_element_type=jnp.float32)
        gn_ref[...] = (hw + b_ref[...]) * dinv_ref[...]

    return pl.pallas_call(
        body,
        out_shape=(
            jax.ShapeDtypeStruct((n, d), jnp.float32),
            jax.ShapeDtypeStruct((n, dh), jnp.float32),
        ),
    )(s0, s1, g_prev, dinv2, w, b2)


def _tc_final(s0, s1, g3, dinv2, h1, h2, att2, wo, bo2):
    n, d = g3.shape
    nc = wo.shape[1]
    isd = 1.0 / math.sqrt(d)

    def body(s0_ref, s1_ref, g_ref, dinv_ref, h1_ref, h2_ref, att_ref, wo_ref,
             bo_ref, o_ref):
        h1 = h1_ref[...]
        h2 = h2_ref[...]
        pre = s0_ref[...] + s1_ref[...] - g_ref[...]
        h3 = jnp.maximum(pre * dinv_ref[...], 0.0)
        a = att_ref[...]
        sc1 = jnp.sum(h1 * a, axis=1, keepdims=True) * isd
        sc2 = jnp.sum(h2 * a, axis=1, keepdims=True) * isd
        sc3 = jnp.sum(h3 * a, axis=1, keepdims=True) * isd
        m = jnp.maximum(sc1, jnp.maximum(sc2, sc3))
        e1 = jnp.exp(sc1 - m)
        e2 = jnp.exp(sc2 - m)
        e3 = jnp.exp(sc3 - m)
        den = e1 + e2 + e3
        hsum = (e1 * h1 + e2 * h2 + e3 * h3) / den
        o_ref[...] = (
            jnp.dot(hsum, wo_ref[...], preferred_element_type=jnp.float32)
            + bo_ref[...]
        )

    return pl.pallas_call(
        body, out_shape=jax.ShapeDtypeStruct((n, nc), jnp.float32)
    )(s0, s1, g3, dinv2, h1, h2, att2, wo, bo2)


def kernel(x, edge_index, W1, b1, W2, b2, W3, b3, att, Wo, bo):
    n = x.shape[0]
    src, dst = edge_index[0], edge_index[1]

    degp = _deg_partials(dst, n)
    deg = jnp.sum(degp, axis=0) + 1.0
    dinv2 = lax.rsqrt(deg)[:, None]

    g1 = _tc_first(x, W1, b1[None, :], dinv2)
    s = _edge_scatter(g1, src, dst)
    h1, g2 = _tc_mid(s[0], s[1], g1, dinv2, W2, b2[None, :])
    s = _edge_scatter(g2, src, dst)
    h2, g3 = _tc_mid(s[0], s[1], g2, dinv2, W3, b3[None, :])
    s = _edge_scatter(g3, src, dst)
    return _tc_final(
        s[0], s[1], g3, dinv2, h1, h2, att[None, :], Wo, bo[None, :]
    )
